# baseline (speedup 1.0000x reference)
import math

import jax
import jax.numpy as jnp
from jax import lax
from jax.experimental import pallas as pl
from jax.experimental.pallas import tpu as pltpu
from jax.experimental.pallas import tpu_sc as plsc

NC = 2
NS = 16
NW = NC * NS
L = 16
CHUNK = 128


def _round_up(a, b):
    return (a + b - 1) // b * b


def _sc_mesh():
    return plsc.VectorSubcoreMesh(core_axis_name="c", subcore_axis_name="s")


def _slab(k):
    for s in range(min(40, k), 1, -2):
        if k % s == 0:
            return s
    return 2


def _deg_body(dst_hbm, deg_hbm, dst_v, hist_v):
    cid = lax.axis_index("c")
    sid = lax.axis_index("s")
    w = sid * NC + cid
    e_per_w = dst_hbm.shape[0] // NW
    n_pad = hist_v.shape[0]

    zeros = jnp.zeros((L,), jnp.float32)

    @pl.loop(0, n_pad, step=L)
    def _(i):
        hist_v[pl.ds(i, L)] = zeros

    pltpu.sync_copy(dst_hbm.at[pl.ds(w * e_per_w, e_per_w)], dst_v)

    ones = jnp.ones((L,), jnp.float32)

    @pl.loop(0, e_per_w, step=L)
    def _(i):
        idx = dst_v[pl.ds(i, L)]
        plsc.addupdate_scatter(hist_v, [idx], ones)

    pltpu.sync_copy(hist_v, deg_hbm.at[w])


def _degree_counts(dst_pad, n_pad):
    e_pad = dst_pad.shape[0]
    kern = pl.kernel(
        _deg_body,
        out_type=jax.ShapeDtypeStruct((NW, n_pad), jnp.float32),
        mesh=_sc_mesh(),
        scratch_types=[
            pltpu.VMEM((e_pad // NW,), jnp.int32),
            pltpu.VMEM((n_pad,), jnp.float32),
        ],
        compiler_params=pltpu.CompilerParams(needs_layout_passes=False),
    )
    return kern(dst_pad)


SHARES = 4
C0_SHARES = 4


def _agg_body(vals_hbm, src_hbm, dst_hbm, out_hbm,
              src_v, dst_v, rows0, rows1, acc_sh, sem0, sem1):
    cid = lax.axis_index("c")
    sid = lax.axis_index("s")
    n_pad, d = acc_sh.shape
    rows_per_sub = n_pad // NS
    slab = src_v.shape[0]
    total_chunks = src_hbm.shape[0]
    k1 = (SHARES - C0_SHARES) * total_chunks // (SHARES * NS)
    k0 = C0_SHARES * total_chunks // (SHARES * NS)
    my_k = jnp.where(cid == 0, k0, k1)
    my_base = jnp.where(cid == 0, sid * k0, NS * k0 + sid * k1)

    zeros = jnp.zeros((L,), jnp.float32)

    @pl.loop(0, CHUNK)
    def _(i):
        @pl.loop(0, d, step=L)
        def _(j):
            rows0[i, pl.ds(j, L)] = zeros

    @pl.loop(0, rows_per_sub, step=CHUNK)
    def _(r):
        pltpu.sync_copy(rows0, acc_sh.at[pl.ds(sid * rows_per_sub + r, CHUNK)])

    plsc.subcore_barrier()

    @pl.loop(0, my_k // slab)
    def _(p):
        base = my_base + p * slab
        pltpu.sync_copy(src_hbm.at[pl.ds(base, slab)], src_v)
        pltpu.sync_copy(dst_hbm.at[pl.ds(base, slab)], dst_v)
        pltpu.async_copy(vals_hbm.at[src_v.at[0]], rows0, sem0)

        @pl.loop(0, slab // 2)
        def _(c2):
            i0 = 2 * c2
            i1 = i0 + 1
            pltpu.async_copy(vals_hbm.at[src_v.at[i1]], rows1, sem1)
            pltpu.make_async_copy(vals_hbm.at[src_v.at[i0]], rows0, sem0).wait()
            pltpu.sync_copy(rows0, acc_sh.at[dst_v.at[i0]], add=True)
            inext = jnp.where(i1 + 1 < slab, i1 + 1, 0)
            pltpu.async_copy(vals_hbm.at[src_v.at[inext]], rows0, sem0)
            pltpu.make_async_copy(vals_hbm.at[src_v.at[i1]], rows1, sem1).wait()
            pltpu.sync_copy(rows1, acc_sh.at[dst_v.at[i1]], add=True)

        pltpu.make_async_copy(vals_hbm.at[src_v.at[0]], rows0, sem0).wait()

    plsc.subcore_barrier()

    r0 = sid * rows_per_sub
    pltpu.sync_copy(acc_sh.at[pl.ds(r0, rows_per_sub)],
                    out_hbm.at[cid, pl.ds(r0, rows_per_sub)])


def _edge_aggregate(vals, src_2d, dst_2d):
    n_pad, d = vals.shape
    total_chunks = src_2d.shape[0]
    k0 = C0_SHARES * total_chunks // (SHARES * NS)
    k1 = (SHARES - C0_SHARES) * total_chunks // (SHARES * NS)
    slab = _slab(k0 if k1 == 0 else math.gcd(k0, k1))
    kern = pl.kernel(
        _agg_body,
        out_type=jax.ShapeDtypeStruct((NC, n_pad, d), jnp.float32),
        mesh=_sc_mesh(),
        scratch_types=[
            pltpu.VMEM((slab, CHUNK), jnp.int32),
            pltpu.VMEM((slab, CHUNK), jnp.int32),
            pltpu.VMEM((CHUNK, d), jnp.float32),
            pltpu.VMEM((CHUNK, d), jnp.float32),
            pltpu.VMEM_SHARED((n_pad, d), jnp.float32),
            pltpu.SemaphoreType.DMA,
            pltpu.SemaphoreType.DMA,
        ],
    )
    return kern(vals, src_2d, dst_2d)


def _prep_body(x_ref, deg_ref, xs_ref, dis_ref):
    ones = jnp.ones((NW, 1), jnp.float32)
    degsum = lax.dot_general(deg_ref[...], ones, (((0,), (0,)), ((), ())),
                             preferred_element_type=jnp.float32)
    dis = lax.rsqrt(degsum + 1.0)
    dis_ref[...] = dis
    xs_ref[...] = x_ref[...] * dis


def _prep(x_pad, deg_parts):
    n_pad, d_in = x_pad.shape
    br = 128
    grid = (n_pad // br,)
    return pl.pallas_call(
        _prep_body,
        grid=grid,
        in_specs=[
            pl.BlockSpec((br, d_in), lambda i: (i, 0)),
            pl.BlockSpec((NW, br), lambda i: (0, i)),
        ],
        out_specs=[
            pl.BlockSpec((br, d_in), lambda i: (i, 0)),
            pl.BlockSpec((br, 1), lambda i: (i, 0)),
        ],
        out_shape=[
            jax.ShapeDtypeStruct((n_pad, d_in), jnp.float32),
            jax.ShapeDtypeStruct((n_pad, 1), jnp.float32),
        ],
    )(x_pad, deg_parts)


def _mid_body(parts_ref, xs_ref, dis_ref, w1_ref, b1_ref, w2_ref, hs2_ref):
    dis = dis_ref[...]
    agg = (parts_ref[0] + parts_ref[1] + xs_ref[...]) * dis
    h1 = jnp.tanh(
        lax.dot_general(agg, w1_ref[...], (((1,), (0,)), ((), ())),
                        preferred_element_type=jnp.float32) + b1_ref[...])
    hs2_ref[...] = lax.dot_general(
        h1, w2_ref[...], (((1,), (0,)), ((), ())),
        preferred_element_type=jnp.float32) * dis


def _mid(parts, xs, dis, w1, b1, w2):
    n_pad, d_in = xs.shape
    d_hid = w1.shape[1]
    d_out = w2.shape[1]
    br = 256
    grid = (n_pad // br,)
    return pl.pallas_call(
        _mid_body,
        grid=grid,
        in_specs=[
            pl.BlockSpec((NC, br, d_in), lambda i: (0, i, 0)),
            pl.BlockSpec((br, d_in), lambda i: (i, 0)),
            pl.BlockSpec((br, 1), lambda i: (i, 0)),
            pl.BlockSpec((d_in, d_hid), lambda i: (0, 0)),
            pl.BlockSpec((1, d_hid), lambda i: (0, 0)),
            pl.BlockSpec((d_hid, d_out), lambda i: (0, 0)),
        ],
        out_specs=pl.BlockSpec((br, d_out), lambda i: (i, 0)),
        out_shape=jax.ShapeDtypeStruct((n_pad, d_out), jnp.float32),
    )(parts, xs, dis, w1, b1, w2)


def _final_body(parts_ref, hs2_ref, dis_ref, b2_ref, out_ref):
    out_ref[...] = ((parts_ref[0] + parts_ref[1] + hs2_ref[...])
                    * dis_ref[...] + b2_ref[...])


def _final(parts, hs2, dis, b2):
    n_pad, d_out = hs2.shape
    br = 512
    grid = (n_pad // br,)
    return pl.pallas_call(
        _final_body,
        grid=grid,
        in_specs=[
            pl.BlockSpec((NC, br, d_out), lambda i: (0, i, 0)),
            pl.BlockSpec((br, d_out), lambda i: (i, 0)),
            pl.BlockSpec((br, 1), lambda i: (i, 0)),
            pl.BlockSpec((1, d_out), lambda i: (0, 0)),
        ],
        out_specs=pl.BlockSpec((br, d_out), lambda i: (i, 0)),
        out_shape=jax.ShapeDtypeStruct((n_pad, d_out), jnp.float32),
    )(parts, hs2, dis, b2)


def kernel(x, edge_index, W1, b1, W2, b2):
    n, d_in = x.shape
    e = edge_index.shape[1]

    n_pad = _round_up(n + 1, NS * CHUNK)
    e_pad = _round_up(e, NS * CHUNK * 2 * SHARES)

    x_pad = jnp.zeros((n_pad, d_in), jnp.float32).at[:n].set(x)
    epad = jnp.full((e_pad - e,), n, jnp.int32)
    src_pad = jnp.concatenate([edge_index[0], epad])
    dst_pad = jnp.concatenate([edge_index[1], epad])
    src_2d = src_pad.reshape(-1, CHUNK)
    dst_2d = dst_pad.reshape(-1, CHUNK)

    deg_parts = _degree_counts(dst_pad, n_pad)
    xs, dis = _prep(x_pad, deg_parts)
    parts1 = _edge_aggregate(xs, src_2d, dst_2d)
    hs2 = _mid(parts1, xs, dis, W1, b1.reshape(1, -1), W2)
    parts2 = _edge_aggregate(hs2, src_2d, dst_2d)
    out_pad = _final(parts2, hs2, dis, b2.reshape(1, -1))
    return out_pad[:n]


# speedup vs baseline: 1.1268x; 1.1268x over previous
import math

import jax
import jax.numpy as jnp
from jax import lax
from jax.experimental import pallas as pl
from jax.experimental.pallas import tpu as pltpu
from jax.experimental.pallas import tpu_sc as plsc

NC = 2
NS = 16
NW = NC * NS
L = 16
CHUNK = 128


def _round_up(a, b):
    return (a + b - 1) // b * b


def _sc_mesh():
    return plsc.VectorSubcoreMesh(core_axis_name="c", subcore_axis_name="s")


def _slab(k):
    for s in range(min(40, k), 1, -2):
        if k % s == 0:
            return s
    return 2


def _deg_body(dst_hbm, deg_hbm, dst_v, hist_v):
    cid = lax.axis_index("c")
    sid = lax.axis_index("s")
    w = sid * NC + cid
    e_per_w = dst_hbm.shape[0] // NW
    n_pad = hist_v.shape[0]

    zeros = jnp.zeros((L,), jnp.float32)

    @pl.loop(0, n_pad, step=L)
    def _(i):
        hist_v[pl.ds(i, L)] = zeros

    pltpu.sync_copy(dst_hbm.at[pl.ds(w * e_per_w, e_per_w)], dst_v)

    ones = jnp.ones((L,), jnp.float32)

    @pl.loop(0, e_per_w, step=L)
    def _(i):
        idx = dst_v[pl.ds(i, L)]
        plsc.addupdate_scatter(hist_v, [idx], ones)

    pltpu.sync_copy(hist_v, deg_hbm.at[w])


def _degree_counts(dst_pad, n_pad):
    e_pad = dst_pad.shape[0]
    kern = pl.kernel(
        _deg_body,
        out_type=jax.ShapeDtypeStruct((NW, n_pad), jnp.float32),
        mesh=_sc_mesh(),
        scratch_types=[
            pltpu.VMEM((e_pad // NW,), jnp.int32),
            pltpu.VMEM((n_pad,), jnp.float32),
        ],
        compiler_params=pltpu.CompilerParams(needs_layout_passes=False),
    )
    return kern(dst_pad)


SHARES = 4
C0_SHARES = 3


def _agg_body(vals_hbm, src_hbm, dst_hbm, out_hbm,
              src_v, dst_v, rows0, rows1, acc_sh, sem0, sem1):
    cid = lax.axis_index("c")
    sid = lax.axis_index("s")
    n_pad, d = acc_sh.shape
    rows_per_sub = n_pad // NS
    slab = src_v.shape[0]
    total_chunks = src_hbm.shape[0]
    k1 = (SHARES - C0_SHARES) * total_chunks // (SHARES * NS)
    k0 = C0_SHARES * total_chunks // (SHARES * NS)
    my_k = jnp.where(cid == 0, k0, k1)
    my_base = jnp.where(cid == 0, sid * k0, NS * k0 + sid * k1)

    zeros = jnp.zeros((L,), jnp.float32)

    @pl.loop(0, CHUNK)
    def _(i):
        @pl.loop(0, d, step=L)
        def _(j):
            rows0[i, pl.ds(j, L)] = zeros

    @pl.loop(0, rows_per_sub, step=CHUNK)
    def _(r):
        pltpu.sync_copy(rows0, acc_sh.at[pl.ds(sid * rows_per_sub + r, CHUNK)])

    plsc.subcore_barrier()

    @pl.loop(0, my_k // slab)
    def _(p):
        base = my_base + p * slab
        pltpu.sync_copy(src_hbm.at[pl.ds(base, slab)], src_v)
        pltpu.sync_copy(dst_hbm.at[pl.ds(base, slab)], dst_v)
        pltpu.async_copy(vals_hbm.at[src_v.at[0]], rows0, sem0)

        @pl.loop(0, slab // 2)
        def _(c2):
            i0 = 2 * c2
            i1 = i0 + 1
            pltpu.async_copy(vals_hbm.at[src_v.at[i1]], rows1, sem1)
            pltpu.make_async_copy(vals_hbm.at[src_v.at[i0]], rows0, sem0).wait()
            pltpu.sync_copy(rows0, acc_sh.at[dst_v.at[i0]], add=True)
            inext = jnp.where(i1 + 1 < slab, i1 + 1, 0)
            pltpu.async_copy(vals_hbm.at[src_v.at[inext]], rows0, sem0)
            pltpu.make_async_copy(vals_hbm.at[src_v.at[i1]], rows1, sem1).wait()
            pltpu.sync_copy(rows1, acc_sh.at[dst_v.at[i1]], add=True)

        pltpu.make_async_copy(vals_hbm.at[src_v.at[0]], rows0, sem0).wait()

    plsc.subcore_barrier()

    r0 = sid * rows_per_sub
    pltpu.sync_copy(acc_sh.at[pl.ds(r0, rows_per_sub)],
                    out_hbm.at[cid, pl.ds(r0, rows_per_sub)])


def _edge_aggregate(vals, src_2d, dst_2d):
    n_pad, d = vals.shape
    total_chunks = src_2d.shape[0]
    k0 = C0_SHARES * total_chunks // (SHARES * NS)
    k1 = (SHARES - C0_SHARES) * total_chunks // (SHARES * NS)
    slab = _slab(k0 if k1 == 0 else math.gcd(k0, k1))
    kern = pl.kernel(
        _agg_body,
        out_type=jax.ShapeDtypeStruct((NC, n_pad, d), jnp.float32),
        mesh=_sc_mesh(),
        scratch_types=[
            pltpu.VMEM((slab, CHUNK), jnp.int32),
            pltpu.VMEM((slab, CHUNK), jnp.int32),
            pltpu.VMEM((CHUNK, d), jnp.float32),
            pltpu.VMEM((CHUNK, d), jnp.float32),
            pltpu.VMEM_SHARED((n_pad, d), jnp.float32),
            pltpu.SemaphoreType.DMA,
            pltpu.SemaphoreType.DMA,
        ],
    )
    return kern(vals, src_2d, dst_2d)


def _prep_body(x_ref, deg_ref, xs_ref, dis_ref):
    ones = jnp.ones((NW, 1), jnp.float32)
    degsum = lax.dot_general(deg_ref[...], ones, (((0,), (0,)), ((), ())),
                             preferred_element_type=jnp.float32)
    dis = lax.rsqrt(degsum + 1.0)
    dis_ref[...] = dis
    xs_ref[...] = x_ref[...] * dis


def _prep(x_pad, deg_parts):
    n_pad, d_in = x_pad.shape
    br = 128
    grid = (n_pad // br,)
    return pl.pallas_call(
        _prep_body,
        grid=grid,
        in_specs=[
            pl.BlockSpec((br, d_in), lambda i: (i, 0)),
            pl.BlockSpec((NW, br), lambda i: (0, i)),
        ],
        out_specs=[
            pl.BlockSpec((br, d_in), lambda i: (i, 0)),
            pl.BlockSpec((br, 1), lambda i: (i, 0)),
        ],
        out_shape=[
            jax.ShapeDtypeStruct((n_pad, d_in), jnp.float32),
            jax.ShapeDtypeStruct((n_pad, 1), jnp.float32),
        ],
    )(x_pad, deg_parts)


def _mid_body(parts_ref, xs_ref, dis_ref, w1_ref, b1_ref, w2_ref, hs2_ref):
    dis = dis_ref[...]
    agg = (parts_ref[0] + parts_ref[1] + xs_ref[...]) * dis
    h1 = jnp.tanh(
        lax.dot_general(agg, w1_ref[...], (((1,), (0,)), ((), ())),
                        preferred_element_type=jnp.float32) + b1_ref[...])
    hs2_ref[...] = lax.dot_general(
        h1, w2_ref[...], (((1,), (0,)), ((), ())),
        preferred_element_type=jnp.float32) * dis


def _mid(parts, xs, dis, w1, b1, w2):
    n_pad, d_in = xs.shape
    d_hid = w1.shape[1]
    d_out = w2.shape[1]
    br = 256
    grid = (n_pad // br,)
    return pl.pallas_call(
        _mid_body,
        grid=grid,
        in_specs=[
            pl.BlockSpec((NC, br, d_in), lambda i: (0, i, 0)),
            pl.BlockSpec((br, d_in), lambda i: (i, 0)),
            pl.BlockSpec((br, 1), lambda i: (i, 0)),
            pl.BlockSpec((d_in, d_hid), lambda i: (0, 0)),
            pl.BlockSpec((1, d_hid), lambda i: (0, 0)),
            pl.BlockSpec((d_hid, d_out), lambda i: (0, 0)),
        ],
        out_specs=pl.BlockSpec((br, d_out), lambda i: (i, 0)),
        out_shape=jax.ShapeDtypeStruct((n_pad, d_out), jnp.float32),
    )(parts, xs, dis, w1, b1, w2)


def _final_body(parts_ref, hs2_ref, dis_ref, b2_ref, out_ref):
    out_ref[...] = ((parts_ref[0] + parts_ref[1] + hs2_ref[...])
                    * dis_ref[...] + b2_ref[...])


def _final(parts, hs2, dis, b2):
    n_pad, d_out = hs2.shape
    br = 512
    grid = (n_pad // br,)
    return pl.pallas_call(
        _final_body,
        grid=grid,
        in_specs=[
            pl.BlockSpec((NC, br, d_out), lambda i: (0, i, 0)),
            pl.BlockSpec((br, d_out), lambda i: (i, 0)),
            pl.BlockSpec((br, 1), lambda i: (i, 0)),
            pl.BlockSpec((1, d_out), lambda i: (0, 0)),
        ],
        out_specs=pl.BlockSpec((br, d_out), lambda i: (i, 0)),
        out_shape=jax.ShapeDtypeStruct((n_pad, d_out), jnp.float32),
    )(parts, hs2, dis, b2)


def kernel(x, edge_index, W1, b1, W2, b2):
    n, d_in = x.shape
    e = edge_index.shape[1]

    n_pad = _round_up(n + 1, NS * CHUNK)
    e_pad = _round_up(e, NS * CHUNK * 2 * SHARES)

    x_pad = jnp.zeros((n_pad, d_in), jnp.float32).at[:n].set(x)
    epad = jnp.full((e_pad - e,), n, jnp.int32)
    src_pad = jnp.concatenate([edge_index[0], epad])
    dst_pad = jnp.concatenate([edge_index[1], epad])
    src_2d = src_pad.reshape(-1, CHUNK)
    dst_2d = dst_pad.reshape(-1, CHUNK)

    deg_parts = _degree_counts(dst_pad, n_pad)
    xs, dis = _prep(x_pad, deg_parts)
    parts1 = _edge_aggregate(xs, src_2d, dst_2d)
    hs2 = _mid(parts1, xs, dis, W1, b1.reshape(1, -1), W2)
    parts2 = _edge_aggregate(hs2, src_2d, dst_2d)
    out_pad = _final(parts2, hs2, dis, b2.reshape(1, -1))
    return out_pad[:n]


# speedup vs baseline: 1.1850x; 1.0516x over previous
import math

import jax
import jax.numpy as jnp
from jax import lax
from jax.experimental import pallas as pl
from jax.experimental.pallas import tpu as pltpu
from jax.experimental.pallas import tpu_sc as plsc

NC = 2
NS = 16
NW = NC * NS
L = 16
CHUNK = 128


def _round_up(a, b):
    return (a + b - 1) // b * b


def _sc_mesh():
    return plsc.VectorSubcoreMesh(core_axis_name="c", subcore_axis_name="s")


def _slab(k):
    for s in range(min(40, k), 1, -2):
        if k % s == 0:
            return s
    return 2


def _deg_body(dst_hbm, deg_hbm, dst_v, hist_v):
    cid = lax.axis_index("c")
    sid = lax.axis_index("s")
    w = sid * NC + cid
    e_per_w = dst_hbm.shape[0] // NW
    n_pad = hist_v.shape[0]

    zeros = jnp.zeros((L,), jnp.float32)

    @pl.loop(0, n_pad, step=L)
    def _(i):
        hist_v[pl.ds(i, L)] = zeros

    pltpu.sync_copy(dst_hbm.at[pl.ds(w * e_per_w, e_per_w)], dst_v)

    ones = jnp.ones((L,), jnp.float32)

    @pl.loop(0, e_per_w, step=L)
    def _(i):
        idx = dst_v[pl.ds(i, L)]
        plsc.addupdate_scatter(hist_v, [idx], ones)

    pltpu.sync_copy(hist_v, deg_hbm.at[w])


def _degree_counts(dst_pad, n_pad):
    e_pad = dst_pad.shape[0]
    kern = pl.kernel(
        _deg_body,
        out_type=jax.ShapeDtypeStruct((NW, n_pad), jnp.float32),
        mesh=_sc_mesh(),
        scratch_types=[
            pltpu.VMEM((e_pad // NW,), jnp.int32),
            pltpu.VMEM((n_pad,), jnp.float32),
        ],
        compiler_params=pltpu.CompilerParams(needs_layout_passes=False),
    )
    return kern(dst_pad)


SHARES = 4
C0_SHARES = 3


def _agg_body(vals_hbm, src_hbm, dst_hbm, out_hbm,
              src_v, dst_v, rows0, rows1, acc_sh, sem0, sem1):
    cid = lax.axis_index("c")
    sid = lax.axis_index("s")
    n_pad, d = acc_sh.shape
    rows_per_sub = n_pad // NS
    slab = src_v.shape[0]
    total_chunks = src_hbm.shape[0]
    k1 = (SHARES - C0_SHARES) * total_chunks // (SHARES * NS)
    k0 = C0_SHARES * total_chunks // (SHARES * NS)
    my_k = jnp.where(cid == 0, k0, k1)
    my_base = jnp.where(cid == 0, sid * k0, NS * k0 + sid * k1)

    zeros = jnp.zeros((L,), jnp.float32)

    @pl.loop(0, CHUNK)
    def _(i):
        @pl.loop(0, d, step=L)
        def _(j):
            rows0[i, pl.ds(j, L)] = zeros

    @pl.loop(0, rows_per_sub, step=CHUNK)
    def _(r):
        pltpu.sync_copy(rows0, acc_sh.at[pl.ds(sid * rows_per_sub + r, CHUNK)])

    plsc.subcore_barrier()

    @pl.loop(0, my_k // slab)
    def _(p):
        base = my_base + p * slab
        pltpu.sync_copy(src_hbm.at[pl.ds(base, slab)], src_v)
        pltpu.sync_copy(dst_hbm.at[pl.ds(base, slab)], dst_v)
        pltpu.async_copy(vals_hbm.at[src_v.at[0]], rows0, sem0)

        @pl.loop(0, slab // 2)
        def _(c2):
            i0 = 2 * c2
            i1 = i0 + 1
            pltpu.async_copy(vals_hbm.at[src_v.at[i1]], rows1, sem1)
            pltpu.make_async_copy(vals_hbm.at[src_v.at[i0]], rows0, sem0).wait()
            pltpu.sync_copy(rows0, acc_sh.at[dst_v.at[i0]], add=True)
            inext = jnp.where(i1 + 1 < slab, i1 + 1, 0)
            pltpu.async_copy(vals_hbm.at[src_v.at[inext]], rows0, sem0)
            pltpu.make_async_copy(vals_hbm.at[src_v.at[i1]], rows1, sem1).wait()
            pltpu.sync_copy(rows1, acc_sh.at[dst_v.at[i1]], add=True)

        pltpu.make_async_copy(vals_hbm.at[src_v.at[0]], rows0, sem0).wait()

    plsc.subcore_barrier()

    r0 = sid * rows_per_sub
    pltpu.sync_copy(acc_sh.at[pl.ds(r0, rows_per_sub)],
                    out_hbm.at[cid, pl.ds(r0, rows_per_sub)])


def _edge_aggregate(vals, src_2d, dst_2d):
    n_pad, d = vals.shape
    total_chunks = src_2d.shape[0]
    k0 = C0_SHARES * total_chunks // (SHARES * NS)
    k1 = (SHARES - C0_SHARES) * total_chunks // (SHARES * NS)
    slab = _slab(k0 if k1 == 0 else math.gcd(k0, k1))
    kern = pl.kernel(
        _agg_body,
        out_type=jax.ShapeDtypeStruct((NC, n_pad, d), jnp.float32),
        mesh=_sc_mesh(),
        scratch_types=[
            pltpu.VMEM((slab, CHUNK), jnp.int32),
            pltpu.VMEM((slab, CHUNK), jnp.int32),
            pltpu.VMEM((CHUNK, d), jnp.float32),
            pltpu.VMEM((CHUNK, d), jnp.float32),
            pltpu.VMEM_SHARED((n_pad, d), jnp.float32),
            pltpu.SemaphoreType.DMA,
            pltpu.SemaphoreType.DMA,
        ],
    )
    return kern(vals, src_2d, dst_2d)


def _prep_body(x_ref, deg_ref, xs_ref, dis_ref):
    ones = jnp.ones((NW, 1), jnp.float32)
    degsum = lax.dot_general(deg_ref[...], ones, (((0,), (0,)), ((), ())),
                             preferred_element_type=jnp.float32)
    dis = lax.rsqrt(degsum + 1.0)
    dis_ref[...] = dis
    xs_ref[...] = x_ref[...] * dis


def _prep(x_pad, deg_parts):
    n_pad, d_in = x_pad.shape
    br = 512
    grid = (n_pad // br,)
    return pl.pallas_call(
        _prep_body,
        grid=grid,
        in_specs=[
            pl.BlockSpec((br, d_in), lambda i: (i, 0)),
            pl.BlockSpec((NW, br), lambda i: (0, i)),
        ],
        out_specs=[
            pl.BlockSpec((br, d_in), lambda i: (i, 0)),
            pl.BlockSpec((br, 1), lambda i: (i, 0)),
        ],
        out_shape=[
            jax.ShapeDtypeStruct((n_pad, d_in), jnp.float32),
            jax.ShapeDtypeStruct((n_pad, 1), jnp.float32),
        ],
    )(x_pad, deg_parts)


def _mid_body(parts_ref, xs_ref, dis_ref, w1_ref, b1_ref, w2_ref, hs2_ref):
    dis = dis_ref[...]
    agg = (parts_ref[0] + parts_ref[1] + xs_ref[...]) * dis
    h1 = jnp.tanh(
        lax.dot_general(agg, w1_ref[...], (((1,), (0,)), ((), ())),
                        preferred_element_type=jnp.float32) + b1_ref[...])
    hs2_ref[...] = lax.dot_general(
        h1, w2_ref[...], (((1,), (0,)), ((), ())),
        preferred_element_type=jnp.float32) * dis


def _mid(parts, xs, dis, w1, b1, w2):
    n_pad, d_in = xs.shape
    d_hid = w1.shape[1]
    d_out = w2.shape[1]
    br = 512
    grid = (n_pad // br,)
    return pl.pallas_call(
        _mid_body,
        grid=grid,
        in_specs=[
            pl.BlockSpec((NC, br, d_in), lambda i: (0, i, 0)),
            pl.BlockSpec((br, d_in), lambda i: (i, 0)),
            pl.BlockSpec((br, 1), lambda i: (i, 0)),
            pl.BlockSpec((d_in, d_hid), lambda i: (0, 0)),
            pl.BlockSpec((1, d_hid), lambda i: (0, 0)),
            pl.BlockSpec((d_hid, d_out), lambda i: (0, 0)),
        ],
        out_specs=pl.BlockSpec((br, d_out), lambda i: (i, 0)),
        out_shape=jax.ShapeDtypeStruct((n_pad, d_out), jnp.float32),
    )(parts, xs, dis, w1, b1, w2)


def _final_body(parts_ref, hs2_ref, dis_ref, b2_ref, out_ref):
    out_ref[...] = ((parts_ref[0] + parts_ref[1] + hs2_ref[...])
                    * dis_ref[...] + b2_ref[...])


def _final(parts, hs2, dis, b2):
    n_pad, d_out = hs2.shape
    br = 512
    grid = (n_pad // br,)
    return pl.pallas_call(
        _final_body,
        grid=grid,
        in_specs=[
            pl.BlockSpec((NC, br, d_out), lambda i: (0, i, 0)),
            pl.BlockSpec((br, d_out), lambda i: (i, 0)),
            pl.BlockSpec((br, 1), lambda i: (i, 0)),
            pl.BlockSpec((1, d_out), lambda i: (0, 0)),
        ],
        out_specs=pl.BlockSpec((br, d_out), lambda i: (i, 0)),
        out_shape=jax.ShapeDtypeStruct((n_pad, d_out), jnp.float32),
    )(parts, hs2, dis, b2)


def kernel(x, edge_index, W1, b1, W2, b2):
    n, d_in = x.shape
    e = edge_index.shape[1]

    n_pad = _round_up(n + 1, NS * CHUNK)
    e_pad = _round_up(e, NS * CHUNK * 2 * SHARES)

    x_pad = jnp.zeros((n_pad, d_in), jnp.float32).at[:n].set(x)
    epad = jnp.full((e_pad - e,), n, jnp.int32)
    src_pad = jnp.concatenate([edge_index[0], epad])
    dst_pad = jnp.concatenate([edge_index[1], epad])
    src_2d = src_pad.reshape(-1, CHUNK)
    dst_2d = dst_pad.reshape(-1, CHUNK)

    deg_parts = _degree_counts(dst_pad, n_pad)
    xs, dis = _prep(x_pad, deg_parts)
    parts1 = _edge_aggregate(xs, src_2d, dst_2d)
    hs2 = _mid(parts1, xs, dis, W1, b1.reshape(1, -1), W2)
    parts2 = _edge_aggregate(hs2, src_2d, dst_2d)
    out_pad = _final(parts2, hs2, dis, b2.reshape(1, -1))
    return out_pad[:n]


# speedup vs baseline: 1.1865x; 1.0013x over previous
import math

import jax
import jax.numpy as jnp
from jax import lax
from jax.experimental import pallas as pl
from jax.experimental.pallas import tpu as pltpu
from jax.experimental.pallas import tpu_sc as plsc

NC = 2
NS = 16
NW = NC * NS
L = 16
CHUNK = 128


def _round_up(a, b):
    return (a + b - 1) // b * b


def _sc_mesh():
    return plsc.VectorSubcoreMesh(core_axis_name="c", subcore_axis_name="s")


def _slab(k):
    for s in range(min(40, k), 1, -2):
        if k % s == 0:
            return s
    return 2


def _deg_body(dst_hbm, deg_hbm, dst_v, hist_v):
    cid = lax.axis_index("c")
    sid = lax.axis_index("s")
    w = sid * NC + cid
    e_per_w = dst_hbm.shape[0] // NW
    n_pad = hist_v.shape[0]

    zeros = jnp.zeros((L,), jnp.float32)

    @pl.loop(0, n_pad, step=L)
    def _(i):
        hist_v[pl.ds(i, L)] = zeros

    pltpu.sync_copy(dst_hbm.at[pl.ds(w * e_per_w, e_per_w)], dst_v)

    ones = jnp.ones((L,), jnp.float32)

    @pl.loop(0, e_per_w, step=L)
    def _(i):
        idx = dst_v[pl.ds(i, L)]
        plsc.addupdate_scatter(hist_v, [idx], ones)

    pltpu.sync_copy(hist_v, deg_hbm.at[w])


def _degree_counts(dst_pad, n_pad):
    e_pad = dst_pad.shape[0]
    kern = pl.kernel(
        _deg_body,
        out_type=jax.ShapeDtypeStruct((NW, n_pad), jnp.float32),
        mesh=_sc_mesh(),
        scratch_types=[
            pltpu.VMEM((e_pad // NW,), jnp.int32),
            pltpu.VMEM((n_pad,), jnp.float32),
        ],
        compiler_params=pltpu.CompilerParams(needs_layout_passes=False),
    )
    return kern(dst_pad)


SHARES = 5
C0_SHARES = 4


def _agg_body(vals_hbm, src_hbm, dst_hbm, out_hbm,
              src_v, dst_v, rows0, rows1, acc_sh, sem0, sem1):
    cid = lax.axis_index("c")
    sid = lax.axis_index("s")
    n_pad, d = acc_sh.shape
    rows_per_sub = n_pad // NS
    slab = src_v.shape[0]
    total_chunks = src_hbm.shape[0]
    k1 = (SHARES - C0_SHARES) * total_chunks // (SHARES * NS)
    k0 = C0_SHARES * total_chunks // (SHARES * NS)
    my_k = jnp.where(cid == 0, k0, k1)
    my_base = jnp.where(cid == 0, sid * k0, NS * k0 + sid * k1)

    zeros = jnp.zeros((L,), jnp.float32)

    @pl.loop(0, CHUNK)
    def _(i):
        @pl.loop(0, d, step=L)
        def _(j):
            rows0[i, pl.ds(j, L)] = zeros

    @pl.loop(0, rows_per_sub, step=CHUNK)
    def _(r):
        pltpu.sync_copy(rows0, acc_sh.at[pl.ds(sid * rows_per_sub + r, CHUNK)])

    plsc.subcore_barrier()

    @pl.loop(0, my_k // slab)
    def _(p):
        base = my_base + p * slab
        pltpu.sync_copy(src_hbm.at[pl.ds(base, slab)], src_v)
        pltpu.sync_copy(dst_hbm.at[pl.ds(base, slab)], dst_v)
        pltpu.async_copy(vals_hbm.at[src_v.at[0]], rows0, sem0)

        @pl.loop(0, slab // 2)
        def _(c2):
            i0 = 2 * c2
            i1 = i0 + 1
            pltpu.async_copy(vals_hbm.at[src_v.at[i1]], rows1, sem1)
            pltpu.make_async_copy(vals_hbm.at[src_v.at[i0]], rows0, sem0).wait()
            pltpu.sync_copy(rows0, acc_sh.at[dst_v.at[i0]], add=True)
            inext = jnp.where(i1 + 1 < slab, i1 + 1, 0)
            pltpu.async_copy(vals_hbm.at[src_v.at[inext]], rows0, sem0)
            pltpu.make_async_copy(vals_hbm.at[src_v.at[i1]], rows1, sem1).wait()
            pltpu.sync_copy(rows1, acc_sh.at[dst_v.at[i1]], add=True)

        pltpu.make_async_copy(vals_hbm.at[src_v.at[0]], rows0, sem0).wait()

    plsc.subcore_barrier()

    r0 = sid * rows_per_sub
    pltpu.sync_copy(acc_sh.at[pl.ds(r0, rows_per_sub)],
                    out_hbm.at[cid, pl.ds(r0, rows_per_sub)])


def _edge_aggregate(vals, src_2d, dst_2d):
    n_pad, d = vals.shape
    total_chunks = src_2d.shape[0]
    k0 = C0_SHARES * total_chunks // (SHARES * NS)
    k1 = (SHARES - C0_SHARES) * total_chunks // (SHARES * NS)
    slab = _slab(k0 if k1 == 0 else math.gcd(k0, k1))
    kern = pl.kernel(
        _agg_body,
        out_type=jax.ShapeDtypeStruct((NC, n_pad, d), jnp.float32),
        mesh=_sc_mesh(),
        scratch_types=[
            pltpu.VMEM((slab, CHUNK), jnp.int32),
            pltpu.VMEM((slab, CHUNK), jnp.int32),
            pltpu.VMEM((CHUNK, d), jnp.float32),
            pltpu.VMEM((CHUNK, d), jnp.float32),
            pltpu.VMEM_SHARED((n_pad, d), jnp.float32),
            pltpu.SemaphoreType.DMA,
            pltpu.SemaphoreType.DMA,
        ],
    )
    return kern(vals, src_2d, dst_2d)


def _prep_body(x_ref, deg_ref, xs_ref, dis_ref):
    ones = jnp.ones((NW, 1), jnp.float32)
    degsum = lax.dot_general(deg_ref[...], ones, (((0,), (0,)), ((), ())),
                             preferred_element_type=jnp.float32)
    dis = lax.rsqrt(degsum + 1.0)
    dis_ref[...] = dis
    xs_ref[...] = x_ref[...] * dis


def _prep(x_pad, deg_parts):
    n_pad, d_in = x_pad.shape
    br = 512
    grid = (n_pad // br,)
    return pl.pallas_call(
        _prep_body,
        grid=grid,
        in_specs=[
            pl.BlockSpec((br, d_in), lambda i: (i, 0)),
            pl.BlockSpec((NW, br), lambda i: (0, i)),
        ],
        out_specs=[
            pl.BlockSpec((br, d_in), lambda i: (i, 0)),
            pl.BlockSpec((br, 1), lambda i: (i, 0)),
        ],
        out_shape=[
            jax.ShapeDtypeStruct((n_pad, d_in), jnp.float32),
            jax.ShapeDtypeStruct((n_pad, 1), jnp.float32),
        ],
    )(x_pad, deg_parts)


def _mid_body(parts_ref, xs_ref, dis_ref, w1_ref, b1_ref, w2_ref, hs2_ref):
    dis = dis_ref[...]
    agg = (parts_ref[0] + parts_ref[1] + xs_ref[...]) * dis
    h1 = jnp.tanh(
        lax.dot_general(agg, w1_ref[...], (((1,), (0,)), ((), ())),
                        preferred_element_type=jnp.float32) + b1_ref[...])
    hs2_ref[...] = lax.dot_general(
        h1, w2_ref[...], (((1,), (0,)), ((), ())),
        preferred_element_type=jnp.float32) * dis


def _mid(parts, xs, dis, w1, b1, w2):
    n_pad, d_in = xs.shape
    d_hid = w1.shape[1]
    d_out = w2.shape[1]
    br = 512
    grid = (n_pad // br,)
    return pl.pallas_call(
        _mid_body,
        grid=grid,
        in_specs=[
            pl.BlockSpec((NC, br, d_in), lambda i: (0, i, 0)),
            pl.BlockSpec((br, d_in), lambda i: (i, 0)),
            pl.BlockSpec((br, 1), lambda i: (i, 0)),
            pl.BlockSpec((d_in, d_hid), lambda i: (0, 0)),
            pl.BlockSpec((1, d_hid), lambda i: (0, 0)),
            pl.BlockSpec((d_hid, d_out), lambda i: (0, 0)),
        ],
        out_specs=pl.BlockSpec((br, d_out), lambda i: (i, 0)),
        out_shape=jax.ShapeDtypeStruct((n_pad, d_out), jnp.float32),
    )(parts, xs, dis, w1, b1, w2)


def _final_body(parts_ref, hs2_ref, dis_ref, b2_ref, out_ref):
    out_ref[...] = ((parts_ref[0] + parts_ref[1] + hs2_ref[...])
                    * dis_ref[...] + b2_ref[...])


def _final(parts, hs2, dis, b2):
    n_pad, d_out = hs2.shape
    br = 512
    grid = (n_pad // br,)
    return pl.pallas_call(
        _final_body,
        grid=grid,
        in_specs=[
            pl.BlockSpec((NC, br, d_out), lambda i: (0, i, 0)),
            pl.BlockSpec((br, d_out), lambda i: (i, 0)),
            pl.BlockSpec((br, 1), lambda i: (i, 0)),
            pl.BlockSpec((1, d_out), lambda i: (0, 0)),
        ],
        out_specs=pl.BlockSpec((br, d_out), lambda i: (i, 0)),
        out_shape=jax.ShapeDtypeStruct((n_pad, d_out), jnp.float32),
    )(parts, hs2, dis, b2)


def kernel(x, edge_index, W1, b1, W2, b2):
    n, d_in = x.shape
    e = edge_index.shape[1]

    n_pad = _round_up(n + 1, NS * CHUNK)
    e_pad = _round_up(e, NS * CHUNK * 2 * SHARES)

    x_pad = jnp.zeros((n_pad, d_in), jnp.float32).at[:n].set(x)
    epad = jnp.full((e_pad - e,), n, jnp.int32)
    src_pad = jnp.concatenate([edge_index[0], epad])
    dst_pad = jnp.concatenate([edge_index[1], epad])
    src_2d = src_pad.reshape(-1, CHUNK)
    dst_2d = dst_pad.reshape(-1, CHUNK)

    deg_parts = _degree_counts(dst_pad, n_pad)
    xs, dis = _prep(x_pad, deg_parts)
    parts1 = _edge_aggregate(xs, src_2d, dst_2d)
    hs2 = _mid(parts1, xs, dis, W1, b1.reshape(1, -1), W2)
    parts2 = _edge_aggregate(hs2, src_2d, dst_2d)
    out_pad = _final(parts2, hs2, dis, b2.reshape(1, -1))
    return out_pad[:n]
